# Initial kernel scaffold; baseline (speedup 1.0000x reference)
import jax
import jax.numpy as jnp
from jax.experimental import pallas as pl


def kernel(label, fixed, table, W, b):
    raise NotImplementedError("write your pallas kernel here")


# speedup vs baseline: 5.5081x; 5.5081x over previous
import functools

import jax
import jax.numpy as jnp
from jax import lax
from jax.experimental import pallas as pl
from jax.experimental.pallas import tpu as pltpu
from jax.experimental.pallas import tpu_sc as plsc

EMBED_N = 100000
HIDDEN = 64
FIXED = 26
BATCH = 4096
SEQ = 50

NC, NS, LANES = 2, 16, 16
NW = NC * NS
NB_W = BATCH // NW
CHL = 2 * SEQ
NCH = NB_W // 2
NVH = HIDDEN // LANES

ROWS_BLK = 4000


def _g_body(t_ref, g_ref):
    x = t_ref[...]
    ssq = jnp.sum(x * x, axis=1, keepdims=True)
    nrm = jnp.sqrt(ssq)
    scale = jnp.where(nrm > 1.0, 1.0 / (nrm + 1e-7), 1.0)
    r = jnp.maximum(x, 0.0) * scale
    g_ref[...] = r * r


def _sc_body(lab_ref, g_ref, out_ref, idx_v, rows_v, stage_v, sem_g, sem_w):
    wid = lax.axis_index("c") * NS + lax.axis_index("s")
    pltpu.sync_copy(lab_ref.at[pl.ds(wid * NCH, NCH)], idx_v)
    pltpu.async_copy(g_ref.at[idx_v.at[0]], rows_v.at[0], sem_g)

    def step(c, p):
        @pl.when(c + 1 < NCH)
        def _():
            pltpu.async_copy(g_ref.at[idx_v.at[c + 1]], rows_v.at[1 - p], sem_g)

        pltpu.make_async_copy(g_ref.at[idx_v.at[c]], rows_v.at[p], sem_g).wait()

        for half in range(2):
            accs = [jnp.zeros((LANES,), jnp.float32) for _ in range(NVH)]
            for l in range(SEQ):
                r = half * SEQ + l
                for j in range(NVH):
                    accs[j] = accs[j] + rows_v[p, r, pl.ds(j * LANES, LANES)]
            for j in range(NVH):
                stage_v[p, half, pl.ds(j * LANES, LANES)] = accs[j]

        @pl.when(c >= 2)
        def _():
            pltpu.make_async_copy(stage_v.at[p], out_ref.at[pl.ds(0, 2)], sem_w).wait()

        b0 = wid * NB_W + 2 * c
        pltpu.async_copy(stage_v.at[p], out_ref.at[pl.ds(b0, 2)], sem_w)

    @pl.loop(0, NCH // 2)
    def _(cc):
        step(2 * cc, 0)
        step(2 * cc + 1, 1)

    pltpu.make_async_copy(stage_v.at[0], out_ref.at[pl.ds(0, 2)], sem_w).wait()
    pltpu.make_async_copy(stage_v.at[1], out_ref.at[pl.ds(0, 2)], sem_w).wait()


_sc_call = functools.partial(
    pl.kernel,
    out_type=jax.ShapeDtypeStruct((BATCH, HIDDEN), jnp.float32),
    mesh=plsc.VectorSubcoreMesh(
        core_axis_name="c", subcore_axis_name="s", num_cores=NC, num_subcores=NS
    ),
    compiler_params=pltpu.CompilerParams(use_tc_tiling_on_sc=False),
    scratch_types=[
        pltpu.VMEM((NCH, CHL), jnp.int32),
        pltpu.VMEM((2, CHL, HIDDEN), jnp.float32),
        pltpu.VMEM((2, 2, HIDDEN), jnp.float32),
        pltpu.SemaphoreType.DMA,
        pltpu.SemaphoreType.DMA,
    ],
)


def _fin_body(acc_ref, fx_ref, w_ref, b_ref, o_ref):
    ew = jnp.sqrt(acc_ref[...])
    w = w_ref[...]
    wf = w[:, :FIXED]
    we = w[:, FIXED:]
    s = (
        jnp.sum(fx_ref[...] * wf, axis=1, keepdims=True)
        + jnp.sum(ew * we, axis=1, keepdims=True)
        + b_ref[0, 0]
    )
    o_ref[...] = jax.nn.sigmoid(s)


@jax.jit
def _impl(label, fixed, table, W, b):
    g = pl.pallas_call(
        _g_body,
        grid=(EMBED_N // ROWS_BLK,),
        in_specs=[pl.BlockSpec((ROWS_BLK, HIDDEN), lambda i: (i, 0))],
        out_specs=pl.BlockSpec((ROWS_BLK, HIDDEN), lambda i: (i, 0)),
        out_shape=jax.ShapeDtypeStruct((EMBED_N, HIDDEN), jnp.float32),
    )(table)
    lab2 = label.astype(jnp.int32).reshape(BATCH * SEQ // CHL, CHL)
    acc = _sc_call(_sc_body)(lab2, g)
    out = pl.pallas_call(
        _fin_body,
        out_shape=jax.ShapeDtypeStruct((BATCH, 1), jnp.float32),
    )(acc, fixed, W, b.reshape(1, 1))
    return out


def kernel(label, fixed, table, W, b):
    return _impl(label, fixed, table, W, b)


# speedup vs baseline: 6.1752x; 1.1211x over previous
import functools

import jax
import jax.numpy as jnp
from jax import lax
from jax.experimental import pallas as pl
from jax.experimental.pallas import tpu as pltpu
from jax.experimental.pallas import tpu_sc as plsc

EMBED_N = 100000
HIDDEN = 64
FIXED = 26
BATCH = 4096
SEQ = 50

NC, NS, LANES = 2, 16, 16
NW = NC * NS
NB_W = BATCH // NW
CHL = 2 * SEQ
NCH = NB_W // 2
NVH = HIDDEN // LANES

ROWS_BLK = 2000


def _g_half(x):
    ssq = jnp.sum(x * x, axis=1, keepdims=True)
    nrm = jnp.sqrt(ssq)
    scale = jnp.where(nrm > 1.0, 1.0 / (nrm + 1e-7), 1.0)
    r = jnp.maximum(x, 0.0) * scale
    return r * r


def _g_body(t_ref, g_ref):
    x = t_ref[...]
    g_ref[...] = jnp.concatenate([_g_half(x[:, :HIDDEN]), _g_half(x[:, HIDDEN:])], axis=1)


def _sc_body(lab_ref, g_ref, out_ref, idx_v, rows_v, stage_v, sem_g, sem_w):
    wid = lax.axis_index("c") * NS + lax.axis_index("s")
    pltpu.sync_copy(lab_ref.at[pl.ds(wid * NCH, NCH)], idx_v)
    pltpu.async_copy(g_ref.at[idx_v.at[0]], rows_v.at[0], sem_g)

    def step(c, p):
        @pl.when(c + 1 < NCH)
        def _():
            pltpu.async_copy(g_ref.at[idx_v.at[c + 1]], rows_v.at[1 - p], sem_g)

        pltpu.make_async_copy(g_ref.at[idx_v.at[c]], rows_v.at[p], sem_g).wait()

        for half in range(2):
            accs = [jnp.zeros((LANES,), jnp.float32) for _ in range(NVH)]
            for l in range(SEQ):
                r = half * SEQ + l
                for j in range(NVH):
                    accs[j] = accs[j] + rows_v[p, r, pl.ds(j * LANES, LANES)]
            for j in range(NVH):
                stage_v[p, half, pl.ds(j * LANES, LANES)] = accs[j]

        @pl.when(c >= 2)
        def _():
            pltpu.make_async_copy(stage_v.at[p], out_ref.at[pl.ds(0, 2)], sem_w).wait()

        b0 = wid * NB_W + 2 * c
        pltpu.async_copy(stage_v.at[p], out_ref.at[pl.ds(b0, 2)], sem_w)

    @pl.loop(0, NCH // 2)
    def _(cc):
        step(2 * cc, 0)
        step(2 * cc + 1, 1)

    pltpu.make_async_copy(stage_v.at[0], out_ref.at[pl.ds(0, 2)], sem_w).wait()
    pltpu.make_async_copy(stage_v.at[1], out_ref.at[pl.ds(0, 2)], sem_w).wait()


_sc_call = functools.partial(
    pl.kernel,
    out_type=jax.ShapeDtypeStruct((BATCH, HIDDEN), jnp.float32),
    mesh=plsc.VectorSubcoreMesh(
        core_axis_name="c", subcore_axis_name="s", num_cores=NC, num_subcores=NS
    ),
    compiler_params=pltpu.CompilerParams(use_tc_tiling_on_sc=False),
    scratch_types=[
        pltpu.VMEM((NCH, CHL), jnp.int32),
        pltpu.VMEM((2, CHL, HIDDEN), jnp.float32),
        pltpu.VMEM((2, 2, HIDDEN), jnp.float32),
        pltpu.SemaphoreType.DMA,
        pltpu.SemaphoreType.DMA,
    ],
)


def _fin_body(acc_ref, fx_ref, w_ref, b_ref, o_ref):
    ew = jnp.sqrt(acc_ref[...])
    w = w_ref[...]
    wf = w[:, :FIXED]
    we = w[:, FIXED:]
    s = (
        jnp.sum(fx_ref[...] * wf, axis=1, keepdims=True)
        + jnp.sum(ew * we, axis=1, keepdims=True)
        + b_ref[0, 0]
    )
    o_ref[...] = jax.nn.sigmoid(s)


@jax.jit
def _impl(label, fixed, table, W, b):
    t2 = table.reshape(EMBED_N // 2, 2 * HIDDEN)
    g2 = pl.pallas_call(
        _g_body,
        grid=(EMBED_N // 2 // ROWS_BLK,),
        in_specs=[pl.BlockSpec((ROWS_BLK, 2 * HIDDEN), lambda i: (i, 0))],
        out_specs=pl.BlockSpec((ROWS_BLK, 2 * HIDDEN), lambda i: (i, 0)),
        out_shape=jax.ShapeDtypeStruct((EMBED_N // 2, 2 * HIDDEN), jnp.float32),
    )(t2)
    g = g2.reshape(EMBED_N, HIDDEN)
    lab2 = label.astype(jnp.int32).reshape(BATCH * SEQ // CHL, CHL)
    acc = _sc_call(_sc_body)(lab2, g)
    out = pl.pallas_call(
        _fin_body,
        out_shape=jax.ShapeDtypeStruct((BATCH, 1), jnp.float32),
    )(acc, fixed, W, b.reshape(1, 1))
    return out


def kernel(label, fixed, table, W, b):
    return _impl(label, fixed, table, W, b)
